# Initial kernel scaffold; baseline (speedup 1.0000x reference)
import jax
import jax.numpy as jnp
from jax.experimental import pallas as pl


def kernel(x1, edge_index1, ent1, batch1, x2, edge_index2, ent2, batch2, atom_emb, gW1, gb1, gW2, gb2, fcW, fcb, ent_emb, eW1, eb1, eW2, eb2, dW1, db1, dW2, db2, dW3, db3):
    raise NotImplementedError("write your pallas kernel here")


# speedup vs baseline: 7.1200x; 7.1200x over previous
import functools

import jax
import jax.numpy as jnp
from jax import lax
from jax.experimental import pallas as pl
from jax.experimental.pallas import tpu as pltpu
from jax.experimental.pallas import tpu_sc as plsc

NCORE = 2
NSUB = 16
NNODE = 10000
NEDGE = 320000
NGRAPH = 512
RB = 1000
NB = NNODE // RB
CH = 80
EPT = NEDGE // NSUB
ZR = NNODE // 10

@functools.cache
def _mesh():
    return plsc.VectorSubcoreMesh(core_axis_name="c", subcore_axis_name="s",
                                  num_cores=NCORE, num_subcores=NSUB)


def _sc_prep_body(dst2_hbm, eidx_hbm, zer_hbm, ones_hbm, etab_hbm,
                  deg_hbm, ent_hbm,
                  idx_v, ones_v, eidx_v, erows_v, acc_sh, sem):
    c = lax.axis_index("c")
    s = lax.axis_index("s")

    @pl.when(s < 10)
    def _():
        pltpu.sync_copy(zer_hbm, acc_sh.at[pl.ds(s * ZR, ZR)])

    pltpu.sync_copy(ones_hbm, ones_v)
    plsc.subcore_barrier()

    def body(k, carry):
        base = c * NEDGE + s * EPT + k * CH
        pltpu.sync_copy(dst2_hbm.at[pl.ds(base, CH)], idx_v)
        pltpu.sync_copy(ones_v, acc_sh.at[idx_v], add=True)
        return carry

    lax.fori_loop(0, EPT // CH, body, 0)
    plsc.subcore_barrier()

    @pl.when(s < 10)
    def _():
        pltpu.sync_copy(acc_sh.at[pl.ds(s * ZR, ZR)],
                        deg_hbm.at[pl.ds(c * NNODE + s * ZR, ZR)])

    w = c * NSUB + s
    pltpu.sync_copy(eidx_hbm.at[pl.ds(w * 32, 32)], eidx_v)
    pltpu.async_copy(etab_hbm.at[eidx_v], erows_v, sem).wait()
    pltpu.sync_copy(erows_v, ent_hbm.at[pl.ds(w * 32, 32)])


@functools.cache
def _sc_prep_call():
    return functools.partial(
        pl.kernel,
        out_type=(jax.ShapeDtypeStruct((NCORE * NNODE, 128), jnp.float32),
                  jax.ShapeDtypeStruct((2 * NGRAPH, 128), jnp.float32)),
        mesh=_mesh(),
        scratch_types=[
            pltpu.VMEM((CH,), jnp.int32),
            pltpu.VMEM((CH, 128), jnp.float32),
            pltpu.VMEM((32,), jnp.int32),
            pltpu.VMEM((32, 128), jnp.float32),
            pltpu.VMEM_SHARED((NNODE, 128), jnp.float32),
            pltpu.SemaphoreType.DMA,
        ],
    )(_sc_prep_body)


def _sc_prep(*args):
    return _sc_prep_call()(*args)


def _sc_scatter_body(u_hbm, src_hbm, dst_hbm, zer128_hbm,
                     sout_hbm,
                     isrc_v, gidx_v, idst_v, rows_v, acc_sh, sem):
    c = lax.axis_index("c")
    s = lax.axis_index("s")

    @pl.when(s < 10)
    def _():
        pltpu.sync_copy(zer128_hbm, acc_sh.at[pl.ds(s * ZR, ZR)])

    plsc.subcore_barrier()
    coff = c * NNODE

    def body(k, carry):
        base = s * EPT + k * CH
        pltpu.sync_copy(src_hbm.at[pl.ds(base, CH)], isrc_v)
        pltpu.sync_copy(dst_hbm.at[pl.ds(base, CH)], idst_v)
        for j in range(CH // 16):
            gidx_v[pl.ds(j * 16, 16)] = isrc_v[pl.ds(j * 16, 16)] + coff
        pltpu.async_copy(u_hbm.at[gidx_v], rows_v, sem).wait()
        pltpu.sync_copy(rows_v, acc_sh.at[idst_v], add=True)
        return carry

    lax.fori_loop(0, EPT // CH, body, 0)
    plsc.subcore_barrier()

    @pl.when(s < 10)
    def _():
        pltpu.sync_copy(acc_sh.at[pl.ds(s * ZR, ZR)],
                        sout_hbm.at[pl.ds(coff + s * ZR, ZR)])


@functools.cache
def _sc_scatter_call():
    return functools.partial(
        pl.kernel,
        out_type=jax.ShapeDtypeStruct((NCORE * NNODE, 128), jnp.float32),
        mesh=_mesh(),
        scratch_types=[
            pltpu.VMEM((CH,), jnp.int32),
            pltpu.VMEM((CH,), jnp.int32),
            pltpu.VMEM((CH,), jnp.int32),
            pltpu.VMEM((CH, 128), jnp.float32),
            pltpu.VMEM_SHARED((NNODE, 128), jnp.float32),
            pltpu.SemaphoreType.DMA,
        ],
    )(_sc_scatter_body)


def _sc_scatter(*args):
    return _sc_scatter_call()(*args)


def _dinv_of(deg_counts):
    return lax.rsqrt(jnp.maximum(deg_counts + 1.0, 1e-12))


def _k1_body(xidx_ref, deg_ref, aemb_ref, gW1_ref, u_ref):
    idx = xidx_ref[0, 0, 0, :]
    dinv = _dinv_of(deg_ref[0, 0, 0, :])
    oh = (idx[:, None] == lax.broadcasted_iota(jnp.int32, (RB, 16), 1)
          ).astype(jnp.float32)
    Wc = jnp.dot(aemb_ref[...], gW1_ref[...],
                 preferred_element_type=jnp.float32)
    u_ref[0] = dinv[:, None] * jnp.dot(oh, Wc,
                                       preferred_element_type=jnp.float32)


def _k2_body(Sa_ref, Sb_ref, ua_ref, ub_ref, deg_ref, gb1_ref, gW2_ref,
             u2_ref):
    dinv = _dinv_of(deg_ref[0, 0, 0, :])
    z = jnp.concatenate([Sa_ref[0] + ua_ref[0], Sb_ref[0] + ub_ref[0]],
                        axis=1)
    z = jnp.maximum(dinv[:, None] * z + gb1_ref[0][None, :], 0.0)
    u2_ref[0] = dinv[:, None] * jnp.dot(z, gW2_ref[...],
                                        preferred_element_type=jnp.float32)


def _k3_body(Sa_ref, Sb_ref, ua_ref, ub_ref, deg_ref, gb2_ref, bat_ref,
             pool_ref):
    i = pl.program_id(1)
    dinv = _dinv_of(deg_ref[0, 0, 0, :])
    h2 = jnp.concatenate([Sa_ref[0] + ua_ref[0], Sb_ref[0] + ub_ref[0]],
                         axis=1)
    h2 = jnp.maximum(dinv[:, None] * h2 + gb2_ref[0][None, :], 0.0)
    bat = bat_ref[0, 0, 0, :]
    ohT = (lax.broadcasted_iota(jnp.int32, (NGRAPH, RB), 0) == bat[None, :]
           ).astype(jnp.float32)
    sums = jnp.dot(ohT, h2, preferred_element_type=jnp.float32)
    cnts = jnp.dot(ohT, jnp.ones((RB, 128), jnp.float32),
                   preferred_element_type=jnp.float32)
    blk = jnp.concatenate([sums, cnts], axis=1)

    @pl.when(i == 0)
    def _():
        pool_ref[0] = blk

    @pl.when(i != 0)
    def _():
        pool_ref[0] = pool_ref[0] + blk


def _k4_body(pool_ref, ent_ref, fcW_ref, fcb_ref, eW1_ref, eb1_ref,
             eW2_ref, eb2_ref, dW1_ref, db1_ref, dW2_ref, db2_ref,
             dW3_ref, db3_ref, out_ref):
    dot = lambda a, b: jnp.dot(a, b, preferred_element_type=jnp.float32)
    egs = None
    for g in range(2):
        pg = pool_ref[g]
        mean = pg[:, :256] / jnp.maximum(pg[:, 256:257], 1.0)
        gf = dot(mean, fcW_ref[...]) + fcb_ref[0][None, :]
        e = jnp.maximum(ent_ref[g], 0.0)
        e = jnp.maximum(dot(e, eW1_ref[...]) + eb1_ref[0][None, :], 0.0)
        e = jnp.maximum(dot(e, eW2_ref[...]) + eb2_ref[0][None, :], 0.0)
        eg = jnp.concatenate([gf, e], axis=1)
        egs = eg if g == 0 else egs + eg
    hh = jnp.maximum(dot(egs, dW1_ref[...]) + db1_ref[0][None, :], 0.0)
    hh = jnp.maximum(dot(hh, dW2_ref[...]) + db2_ref[0][None, :], 0.0)
    out_ref[...] = dot(hh, dW3_ref[...]) + db3_ref[0][None, :]


def _meta_spec():
    return pl.BlockSpec((1, 1, 1, RB), lambda g, h, i: (g, i, 0, 0))


def _meta_spec2():
    return pl.BlockSpec((1, 1, 1, RB), lambda g, i: (g, i, 0, 0))


def _half_view(h_fixed=None):
    if h_fixed is None:
        return pl.BlockSpec((1, RB, 128), lambda g, h, i: (g, h * NB + i, 0))
    return pl.BlockSpec((1, RB, 128),
                        lambda g, i, _h=h_fixed: (g, _h * NB + i, 0))


def kernel(x1, edge_index1, ent1, batch1, x2, edge_index2, ent2, batch2,
           atom_emb, gW1, gb1, gW2, gb2, fcW, fcb,
           ent_emb, eW1, eb1, eW2, eb2,
           dW1, db1, dW2, db2, dW3, db3):
    f32 = jnp.float32
    i32 = jnp.int32

    dst2 = jnp.concatenate([edge_index1[1], edge_index2[1]]).astype(i32)
    eidx = jnp.concatenate([ent1, ent2]).astype(i32)
    zer128 = jnp.zeros((ZR, 128), f32)
    ones128 = jnp.ones((CH, 128), f32)

    deg3, ent_rows = _sc_prep(dst2, eidx, zer128, ones128, ent_emb)
    deg4 = deg3[:, 0].reshape(2, NB, 1, RB)

    xidx4 = jnp.stack([x1, x2]).astype(i32).reshape(2, NB, 1, RB)
    bat4 = jnp.stack([batch1, batch2]).astype(i32).reshape(2, NB, 1, RB)
    aemb_p = jnp.concatenate([atom_emb, jnp.zeros((5, 128), f32)], axis=0)

    u1 = pl.pallas_call(
        _k1_body,
        grid=(2, 2, NB),
        in_specs=[
            _meta_spec(), _meta_spec(),
            pl.BlockSpec((16, 128), lambda g, h, i: (0, 0)),
            pl.BlockSpec((128, 128), lambda g, h, i: (0, h)),
        ],
        out_specs=_half_view(),
        out_shape=jax.ShapeDtypeStruct((2, 2 * NNODE, 128), f32),
    )(xidx4, deg4, aemb_p, gW1)

    def conv_scatter(u):
        s0 = _sc_scatter(u[0], edge_index1[0].astype(i32),
                         edge_index1[1].astype(i32), zer128)
        s1 = _sc_scatter(u[1], edge_index2[0].astype(i32),
                         edge_index2[1].astype(i32), zer128)
        return jnp.stack([s0, s1])

    S1 = conv_scatter(u1)

    gb1r = gb1.reshape(1, 256)
    u2 = pl.pallas_call(
        _k2_body,
        grid=(2, 2, NB),
        in_specs=[
            pl.BlockSpec((1, RB, 128), lambda g, h, i: (g, i, 0)),
            pl.BlockSpec((1, RB, 128), lambda g, h, i: (g, NB + i, 0)),
            pl.BlockSpec((1, RB, 128), lambda g, h, i: (g, i, 0)),
            pl.BlockSpec((1, RB, 128), lambda g, h, i: (g, NB + i, 0)),
            _meta_spec(),
            pl.BlockSpec((1, 256), lambda g, h, i: (0, 0)),
            pl.BlockSpec((256, 128), lambda g, h, i: (0, h)),
        ],
        out_specs=_half_view(),
        out_shape=jax.ShapeDtypeStruct((2, 2 * NNODE, 128), f32),
    )(S1, S1, u1, u1, deg4, gb1r, gW2)

    S2 = conv_scatter(u2)

    gb2r = gb2.reshape(1, 256)
    pooled = pl.pallas_call(
        _k3_body,
        grid=(2, NB),
        in_specs=[
            pl.BlockSpec((1, RB, 128), lambda g, i: (g, i, 0)),
            pl.BlockSpec((1, RB, 128), lambda g, i: (g, NB + i, 0)),
            pl.BlockSpec((1, RB, 128), lambda g, i: (g, i, 0)),
            pl.BlockSpec((1, RB, 128), lambda g, i: (g, NB + i, 0)),
            _meta_spec2(),
            pl.BlockSpec((1, 256), lambda g, i: (0, 0)),
            _meta_spec2(),
        ],
        out_specs=pl.BlockSpec((1, NGRAPH, 384), lambda g, i: (g, 0, 0)),
        out_shape=jax.ShapeDtypeStruct((2, NGRAPH, 384), f32),
    )(S2, S2, u2, u2, deg4, gb2r, bat4)

    ent3 = ent_rows.reshape(2, NGRAPH, 128)
    full = lambda *s: pl.BlockSpec(s, lambda: tuple(0 for _ in s))
    out = pl.pallas_call(
        _k4_body,
        grid=(),
        in_specs=[
            full(2, NGRAPH, 384), full(2, NGRAPH, 128),
            full(256, 256), full(1, 256),
            full(128, 256), full(1, 256), full(256, 256), full(1, 256),
            full(512, 512), full(1, 512), full(512, 512), full(1, 512),
            full(512, 128), full(1, 128),
        ],
        out_specs=full(NGRAPH, 128),
        out_shape=jax.ShapeDtypeStruct((NGRAPH, 128), f32),
    )(pooled, ent3, fcW, fcb.reshape(1, 256),
      eW1, eb1.reshape(1, 256), eW2, eb2.reshape(1, 256),
      dW1, db1.reshape(1, 512), dW2, db2.reshape(1, 512),
      dW3, db3.reshape(1, 128))
    return out
